# baseline (speedup 1.0000x reference)
import functools

import jax
import jax.numpy as jnp
from jax import lax
from jax.experimental import pallas as pl
from jax.experimental.pallas import tpu as pltpu
from jax.experimental.pallas import tpu_sc as plsc

_VOCAB = 1000000
_HID = 64
_BATCH = 4096
_HIST = 200
_NUM_IDX = _BATCH * _HIST
_SPLITS = ((0, 104), (104, 96))
_NC, _NS = 2, 16
_NW = _NC * _NS
_ROWS_W = _BATCH // _NW
_BB = 8
_BROWS = _BB * _HIST
_NBURST = _ROWS_W // _BB
_NBUF = 2


def _gather_body(idx_hbm, tab_hbm, out_hbm, idx_v, rows_v, isem, gsem, wsem):
    wid = lax.axis_index("s") * _NC + lax.axis_index("c")
    base = wid * _ROWS_W

    def idx_copy(b, bb0):
        off = pl.multiple_of(base + bb0, _BB)
        return pltpu.make_async_copy(
            idx_hbm.at[pl.ds(off, _BB)], idx_v.at[b], isem.at[b]
        )

    def wb_copy(b, bb0):
        off = pl.multiple_of((base + bb0) * _HIST, _BROWS)
        return pltpu.make_async_copy(
            rows_v.at[b], out_hbm.at[pl.ds(off, _BROWS)], wsem.at[b]
        )

    def gather_copy(b, r, half):
        lo, n = _SPLITS[half]
        return pltpu.make_async_copy(
            tab_hbm.at[idx_v.at[b, r, pl.ds(lo, n)]],
            rows_v.at[b, pl.ds(r * _HIST + lo, n)],
            gsem,
        )

    for b in range(_NBUF):
        idx_copy(b, b * _BB).start()

    def burst_pair(i, carry):
        for b in range(_NBUF):
            bb0 = (i * _NBUF + b) * _BB
            @pl.when(i > 0)
            def _():
                wb_copy(b, bb0).wait()

            idx_copy(b, bb0).wait()
            for r in range(_BB):
                gather_copy(b, r, 0).start()
                gather_copy(b, r, 1).start()
            for r in range(_BB):
                gather_copy(b, r, 0).wait()
                gather_copy(b, r, 1).wait()
            @pl.when(i + 1 < _NBURST // _NBUF)
            def _():
                idx_copy(b, bb0 + _NBUF * _BB).start()

            wb_copy(b, bb0).start()
        return carry

    lax.fori_loop(0, _NBURST // _NBUF, burst_pair, 0)
    last = (_NBURST - _NBUF) * _BB
    for b in range(_NBUF):
        wb_copy(b, last + b * _BB).wait()


_gather = pl.kernel(
    _gather_body,
    out_type=jax.ShapeDtypeStruct((_NUM_IDX, _HID), jnp.bfloat16),
    mesh=plsc.VectorSubcoreMesh(core_axis_name="c", subcore_axis_name="s"),
    scratch_types=[
        pltpu.VMEM((_NBUF, _BB, _HIST), jnp.int32),
        pltpu.VMEM((_NBUF, _BROWS, _HID), jnp.bfloat16),
        pltpu.SemaphoreType.DMA((_NBUF,)),
        pltpu.SemaphoreType.DMA,
        pltpu.SemaphoreType.DMA((_NBUF,)),
    ],
    compiler_params=pltpu.CompilerParams(use_tc_tiling_on_sc=False),
)


_PB = 32
_BLK = _PB * _HIST


def _proj_body(h_ref, w_ref, b_ref, out_ref):
    acc = lax.dot_general(
        h_ref[...], w_ref[...], (((1,), (1,)), ((), ())),
        preferred_element_type=jnp.float32,
    )
    acc = acc + b_ref[...].astype(jnp.float32)
    out_ref[...] = acc.astype(jnp.bfloat16).reshape(_PB, _HIST, _HID)


_proj = pl.pallas_call(
    _proj_body,
    grid=(_NUM_IDX // _BLK,),
    in_specs=[
        pl.BlockSpec((_BLK, _HID), lambda i: (i, 0)),
        pl.BlockSpec((_HID, _HID), lambda i: (0, 0)),
        pl.BlockSpec((1, _HID), lambda i: (0, 0)),
    ],
    out_specs=pl.BlockSpec((_PB, _HIST, _HID), lambda i: (i, 0, 0)),
    out_shape=jax.ShapeDtypeStruct((_BATCH, _HIST, _HID), jnp.bfloat16),
)


def kernel(x, table, W, b):
    h = _gather(x.astype(jnp.int32), table)
    return _proj(h, W, b.reshape(1, _HID))


# speedup vs baseline: 1.0943x; 1.0943x over previous
import functools

import jax
import jax.numpy as jnp
from jax import lax
from jax.experimental import pallas as pl
from jax.experimental.pallas import tpu as pltpu
from jax.experimental.pallas import tpu_sc as plsc

_VOCAB = 1000000
_HID = 64
_BATCH = 4096
_HIST = 200
_NUM_IDX = _BATCH * _HIST
_SPLITS = ((0, 104), (104, 96))
_NC, _NS = 2, 16
_NW = _NC * _NS
_ROWS_W = _BATCH // _NW
_BB = 8
_BROWS = _BB * _HIST
_NBURST = _ROWS_W // _BB
_NBUF = 2


def _gather_body(idx_hbm, tab_hbm, out_hbm, idx_v, rows_v, isem, gsem, wsem):
    wid = lax.axis_index("s") * _NC + lax.axis_index("c")
    base = wid * _ROWS_W

    def idx_copy(b, bb0):
        off = pl.multiple_of(base + bb0, _BB)
        return pltpu.make_async_copy(
            idx_hbm.at[pl.ds(off, _BB)], idx_v.at[b], isem.at[b]
        )

    def wb_copy(b, bb0):
        off = pl.multiple_of((base + bb0) * _HIST, _BROWS)
        return pltpu.make_async_copy(
            rows_v.at[b], out_hbm.at[pl.ds(off, _BROWS)], wsem.at[b]
        )

    def gather_copy(b, r, half):
        lo, n = _SPLITS[half]
        return pltpu.make_async_copy(
            tab_hbm.at[idx_v.at[b, r, pl.ds(lo, n)]],
            rows_v.at[b, pl.ds(r * _HIST + lo, n)],
            gsem,
        )

    for b in range(_NBUF):
        idx_copy(b, b * _BB).start()

    def burst_pair(i, carry):
        for b in range(_NBUF):
            bb0 = (i * _NBUF + b) * _BB
            @pl.when(i > 0)
            def _():
                wb_copy(b, bb0).wait()

            idx_copy(b, bb0).wait()
            for r in range(_BB):
                gather_copy(b, r, 0).start()
                gather_copy(b, r, 1).start()
            for r in range(_BB):
                gather_copy(b, r, 0).wait()
                gather_copy(b, r, 1).wait()
            @pl.when(i + 1 < _NBURST // _NBUF)
            def _():
                idx_copy(b, bb0 + _NBUF * _BB).start()

            wb_copy(b, bb0).start()
        return carry

    lax.fori_loop(0, _NBURST // _NBUF, burst_pair, 0)
    last = (_NBURST - _NBUF) * _BB
    for b in range(_NBUF):
        wb_copy(b, last + b * _BB).wait()


_gather = pl.kernel(
    _gather_body,
    out_type=jax.ShapeDtypeStruct((_NUM_IDX, _HID), jnp.bfloat16),
    mesh=plsc.VectorSubcoreMesh(core_axis_name="c", subcore_axis_name="s"),
    scratch_types=[
        pltpu.VMEM((_NBUF, _BB, _HIST), jnp.int32),
        pltpu.VMEM((_NBUF, _BROWS, _HID), jnp.bfloat16),
        pltpu.SemaphoreType.DMA((_NBUF,)),
        pltpu.SemaphoreType.DMA,
        pltpu.SemaphoreType.DMA((_NBUF,)),
    ],
    compiler_params=pltpu.CompilerParams(use_tc_tiling_on_sc=False),
)


_BLK = 8192


def _proj_body(h_ref, w_ref, b_ref, out_ref):
    acc = lax.dot_general(
        h_ref[...], w_ref[...], (((1,), (1,)), ((), ())),
        preferred_element_type=jnp.float32,
    )
    out_ref[...] = (acc + b_ref[...].astype(jnp.float32)).astype(jnp.bfloat16)


_proj = pl.pallas_call(
    _proj_body,
    grid=(_NUM_IDX // _BLK,),
    in_specs=[
        pl.BlockSpec((_BLK, _HID), lambda i: (i, 0)),
        pl.BlockSpec((_HID, _HID), lambda i: (0, 0)),
        pl.BlockSpec((1, _HID), lambda i: (0, 0)),
    ],
    out_specs=pl.BlockSpec((_BLK, _HID), lambda i: (i, 0)),
    out_shape=jax.ShapeDtypeStruct((_NUM_IDX, _HID), jnp.bfloat16),
)


def kernel(x, table, W, b):
    h = _gather(x.astype(jnp.int32), table)
    out = _proj(h, W, b.reshape(1, _HID))
    return out.reshape(_BATCH, _HIST, _HID)


# speedup vs baseline: 1.1302x; 1.0327x over previous
import jax
import jax.numpy as jnp
from jax import lax
from jax.experimental import pallas as pl
from jax.experimental.pallas import tpu as pltpu
from jax.experimental.pallas import tpu_sc as plsc

_VOCAB = 1000000
_HID = 64
_BATCH = 4096
_HIST = 200
_NUM_IDX = _BATCH * _HIST
_SPLITS = ((0, 104), (104, 96))
_NC, _NS = 2, 16
_NW = _NC * _NS
_ROWS_W = _BATCH // _NW
_BB = 8
_BROWS = _BB * _HIST
_NBURST = _ROWS_W // _BB
_NBUF = 2


def _gather_body(idx_hbm, tab_hbm, out_hbm, idx_v, rows_v, isem, gsem, wsem):
    wid = lax.axis_index("s") * _NC + lax.axis_index("c")
    base = wid * _ROWS_W

    def idx_copy(b, bb0):
        off = pl.multiple_of(base + bb0, _BB)
        return pltpu.make_async_copy(
            idx_hbm.at[pl.ds(off, _BB)], idx_v.at[b], isem.at[b]
        )

    def wb_copy(b, bb0):
        off = pl.multiple_of((base + bb0) * _HIST, _BROWS)
        return pltpu.make_async_copy(
            rows_v.at[b], out_hbm.at[pl.ds(off, _BROWS)], wsem.at[b]
        )

    def gather_copy(b, r, half):
        lo, n = _SPLITS[half]
        return pltpu.make_async_copy(
            tab_hbm.at[idx_v.at[b, r, pl.ds(lo, n)]],
            rows_v.at[b, pl.ds(r * _HIST + lo, n)],
            gsem,
        )

    for b in range(_NBUF):
        idx_copy(b, b * _BB).start()

    def burst_pair(i, carry):
        for b in range(_NBUF):
            bb0 = (i * _NBUF + b) * _BB
            @pl.when(i > 0)
            def _():
                wb_copy(b, bb0).wait()

            idx_copy(b, bb0).wait()
            for r in range(_BB):
                gather_copy(b, r, 0).start()
                gather_copy(b, r, 1).start()
            for r in range(_BB):
                gather_copy(b, r, 0).wait()
                gather_copy(b, r, 1).wait()
            @pl.when(i + 1 < _NBURST // _NBUF)
            def _():
                idx_copy(b, bb0 + _NBUF * _BB).start()

            wb_copy(b, bb0).start()
        return carry

    lax.fori_loop(0, _NBURST // _NBUF, burst_pair, 0)
    last = (_NBURST - _NBUF) * _BB
    for b in range(_NBUF):
        wb_copy(b, last + b * _BB).wait()


_gather = pl.kernel(
    _gather_body,
    out_type=jax.ShapeDtypeStruct((_NUM_IDX, _HID), jnp.bfloat16),
    mesh=plsc.VectorSubcoreMesh(core_axis_name="c", subcore_axis_name="s"),
    scratch_types=[
        pltpu.VMEM((_NBUF, _BB, _HIST), jnp.int32),
        pltpu.VMEM((_NBUF, _BROWS, _HID), jnp.bfloat16),
        pltpu.SemaphoreType.DMA((_NBUF,)),
        pltpu.SemaphoreType.DMA,
        pltpu.SemaphoreType.DMA((_NBUF,)),
    ],
    compiler_params=pltpu.CompilerParams(use_tc_tiling_on_sc=False),
)


_PK = 4
_NPACK = _NUM_IDX // _PK
_BLKP = 2048


def _proj_body(h_ref, w_ref, b_ref, out_ref):
    acc = lax.dot_general(
        h_ref[...], w_ref[...], (((1,), (0,)), ((), ())),
        preferred_element_type=jnp.float32,
    )
    out_ref[...] = (acc + b_ref[...].astype(jnp.float32)).astype(jnp.bfloat16)


_proj = pl.pallas_call(
    _proj_body,
    grid=(_NPACK // _BLKP,),
    in_specs=[
        pl.BlockSpec((_BLKP, _PK * _HID), lambda i: (i, 0)),
        pl.BlockSpec((_PK * _HID, _PK * _HID), lambda i: (0, 0)),
        pl.BlockSpec((1, _PK * _HID), lambda i: (0, 0)),
    ],
    out_specs=pl.BlockSpec((_BLKP, _PK * _HID), lambda i: (i, 0)),
    out_shape=jax.ShapeDtypeStruct((_NPACK, _PK * _HID), jnp.bfloat16),
)


def kernel(x, table, W, b):
    h = _gather(x.astype(jnp.int32), table)
    hp = h.reshape(_NPACK, _PK * _HID)
    wt = W.T
    z = jnp.zeros_like(wt)
    w4 = jnp.block(
        [[wt if i == j else z for j in range(_PK)] for i in range(_PK)]
    )
    b4 = jnp.tile(b, _PK).reshape(1, _PK * _HID)
    out = _proj(hp, w4, b4)
    return out.reshape(_BATCH, _HIST, _HID)


# speedup vs baseline: 1.1565x; 1.0233x over previous
import jax
import jax.numpy as jnp
from jax import lax
from jax.experimental import pallas as pl
from jax.experimental.pallas import tpu as pltpu
from jax.experimental.pallas import tpu_sc as plsc

_VOCAB = 1000000
_HID = 64
_BATCH = 4096
_HIST = 200
_NUM_IDX = _BATCH * _HIST
_SPLITS = ((0, 104), (104, 96))
_NC, _NS = 2, 16
_NW = _NC * _NS
_ROWS_W = _BATCH // _NW
_BB = 8
_BROWS = _BB * _HIST
_NBURST = _ROWS_W // _BB
_NBUF = 2


def _gather_body(idx_hbm, tab_hbm, out_hbm, idx_v, rows_v, isem, gsem, wsem):
    wid = lax.axis_index("s") * _NC + lax.axis_index("c")
    base = wid * _ROWS_W

    def idx_copy(b, bb0):
        off = pl.multiple_of(base + bb0, _BB)
        return pltpu.make_async_copy(
            idx_hbm.at[pl.ds(off, _BB)], idx_v.at[b], isem.at[b]
        )

    def wb_copy(b, bb0):
        off = pl.multiple_of((base + bb0) * _HIST, _BROWS)
        return pltpu.make_async_copy(
            rows_v.at[b], out_hbm.at[pl.ds(off, _BROWS)], wsem.at[b]
        )

    def gather_copy(b, r, half):
        lo, n = _SPLITS[half]
        return pltpu.make_async_copy(
            tab_hbm.at[idx_v.at[b, r, pl.ds(lo, n)]],
            rows_v.at[b, pl.ds(r * _HIST + lo, n)],
            gsem,
        )

    for b in range(_NBUF):
        idx_copy(b, b * _BB).start()

    def burst_pair(i, carry):
        for b in range(_NBUF):
            bb0 = (i * _NBUF + b) * _BB
            @pl.when(i > 0)
            def _():
                wb_copy(b, bb0).wait()

            idx_copy(b, bb0).wait()
            for r in range(_BB):
                gather_copy(b, r, 0).start()
                gather_copy(b, r, 1).start()
            for r in range(_BB):
                gather_copy(b, r, 0).wait()
                gather_copy(b, r, 1).wait()
            @pl.when(i + 1 < _NBURST // _NBUF)
            def _():
                idx_copy(b, bb0 + _NBUF * _BB).start()

            wb_copy(b, bb0).start()
        return carry

    lax.fori_loop(0, _NBURST // _NBUF, burst_pair, 0)
    last = (_NBURST - _NBUF) * _BB
    for b in range(_NBUF):
        wb_copy(b, last + b * _BB).wait()


_gather = pl.kernel(
    _gather_body,
    out_type=jax.ShapeDtypeStruct((_NUM_IDX, _HID), jnp.bfloat16),
    mesh=plsc.VectorSubcoreMesh(core_axis_name="c", subcore_axis_name="s"),
    scratch_types=[
        pltpu.VMEM((_NBUF, _BB, _HIST), jnp.int32),
        pltpu.VMEM((_NBUF, _BROWS, _HID), jnp.bfloat16),
        pltpu.SemaphoreType.DMA((_NBUF,)),
        pltpu.SemaphoreType.DMA,
        pltpu.SemaphoreType.DMA((_NBUF,)),
    ],
    compiler_params=pltpu.CompilerParams(use_tc_tiling_on_sc=False),
)


_PK = 4
_NPACK = _NUM_IDX // _PK
_BLKP = 4096


def _proj_body(h_ref, w_ref, b_ref, out_ref):
    acc = lax.dot_general(
        h_ref[...], w_ref[...], (((1,), (0,)), ((), ())),
        preferred_element_type=jnp.float32,
    )
    out_ref[...] = (acc + b_ref[...].astype(jnp.float32)).astype(jnp.bfloat16)


_proj = pl.pallas_call(
    _proj_body,
    grid=(_NPACK // _BLKP,),
    in_specs=[
        pl.BlockSpec((_BLKP, _PK * _HID), lambda i: (i, 0)),
        pl.BlockSpec((_PK * _HID, _PK * _HID), lambda i: (0, 0)),
        pl.BlockSpec((1, _PK * _HID), lambda i: (0, 0)),
    ],
    out_specs=pl.BlockSpec((_BLKP, _PK * _HID), lambda i: (i, 0)),
    out_shape=jax.ShapeDtypeStruct((_NPACK, _PK * _HID), jnp.bfloat16),
)


def kernel(x, table, W, b):
    h = _gather(x.astype(jnp.int32), table)
    hp = h.reshape(_NPACK, _PK * _HID)
    wt = W.T
    z = jnp.zeros_like(wt)
    w4 = jnp.block(
        [[wt if i == j else z for j in range(_PK)] for i in range(_PK)]
    )
    b4 = jnp.tile(b, _PK).reshape(1, _PK * _HID)
    out = _proj(hp, w4, b4)
    return out.reshape(_BATCH, _HIST, _HID)


# speedup vs baseline: 1.1639x; 1.0064x over previous
import jax
import jax.numpy as jnp
from jax import lax
from jax.experimental import pallas as pl
from jax.experimental.pallas import tpu as pltpu
from jax.experimental.pallas import tpu_sc as plsc

_VOCAB = 1000000
_HID = 64
_BATCH = 4096
_HIST = 200
_NUM_IDX = _BATCH * _HIST
_SPLITS = ((0, 104), (104, 96))
_NC, _NS = 2, 16
_NW = _NC * _NS
_ROWS_W = _BATCH // _NW
_BB = 8
_BROWS = _BB * _HIST
_NBURST = _ROWS_W // _BB
_NBUF = 2


def _gather_body(idx_hbm, tab_hbm, out_hbm, idx_v, rows_v, isem, gsem, wsem):
    wid = lax.axis_index("s") * _NC + lax.axis_index("c")
    base = wid * _ROWS_W

    def idx_copy(b, bb0):
        off = pl.multiple_of(base + bb0, _BB)
        return pltpu.make_async_copy(
            idx_hbm.at[pl.ds(off, _BB)], idx_v.at[b], isem.at[b]
        )

    def wb_copy(b, bb0):
        off = pl.multiple_of((base + bb0) * _HIST, _BROWS)
        return pltpu.make_async_copy(
            rows_v.at[b], out_hbm.at[pl.ds(off, _BROWS)], wsem.at[b]
        )

    def gather_copy(b, r, half):
        lo, n = _SPLITS[half]
        return pltpu.make_async_copy(
            tab_hbm.at[idx_v.at[b, r, pl.ds(lo, n)]],
            rows_v.at[b, pl.ds(r * _HIST + lo, n)],
            gsem,
        )

    for b in range(_NBUF):
        idx_copy(b, b * _BB).start()

    def burst_pair(i, carry):
        for b in range(_NBUF):
            bb0 = (i * _NBUF + b) * _BB
            @pl.when(i > 0)
            def _():
                wb_copy(b, bb0).wait()

            idx_copy(b, bb0).wait()
            for r in range(_BB):
                gather_copy(b, r, 0).start()
                gather_copy(b, r, 1).start()
            for r in range(_BB):
                gather_copy(b, r, 0).wait()
                gather_copy(b, r, 1).wait()
            @pl.when(i + 1 < _NBURST // _NBUF)
            def _():
                idx_copy(b, bb0 + _NBUF * _BB).start()

            wb_copy(b, bb0).start()
        return carry

    lax.fori_loop(0, _NBURST // _NBUF, burst_pair, 0)
    last = (_NBURST - _NBUF) * _BB
    for b in range(_NBUF):
        wb_copy(b, last + b * _BB).wait()


_gather = pl.kernel(
    _gather_body,
    out_type=jax.ShapeDtypeStruct((_NUM_IDX, _HID), jnp.bfloat16),
    mesh=plsc.VectorSubcoreMesh(core_axis_name="c", subcore_axis_name="s"),
    scratch_types=[
        pltpu.VMEM((_NBUF, _BB, _HIST), jnp.int32),
        pltpu.VMEM((_NBUF, _BROWS, _HID), jnp.bfloat16),
        pltpu.SemaphoreType.DMA((_NBUF,)),
        pltpu.SemaphoreType.DMA,
        pltpu.SemaphoreType.DMA((_NBUF,)),
    ],
    compiler_params=pltpu.CompilerParams(use_tc_tiling_on_sc=False),
)


_PK = 4
_NPACK = _NUM_IDX // _PK
_BLKP = 8192


def _proj_body(h_ref, w_ref, b_ref, out_ref):
    acc = lax.dot_general(
        h_ref[...], w_ref[...], (((1,), (0,)), ((), ())),
        preferred_element_type=jnp.float32,
    )
    out_ref[...] = (acc + b_ref[...].astype(jnp.float32)).astype(jnp.bfloat16)


_proj = pl.pallas_call(
    _proj_body,
    grid=(_NPACK // _BLKP,),
    in_specs=[
        pl.BlockSpec((_BLKP, _PK * _HID), lambda i: (i, 0)),
        pl.BlockSpec((_PK * _HID, _PK * _HID), lambda i: (0, 0)),
        pl.BlockSpec((1, _PK * _HID), lambda i: (0, 0)),
    ],
    out_specs=pl.BlockSpec((_BLKP, _PK * _HID), lambda i: (i, 0)),
    out_shape=jax.ShapeDtypeStruct((_NPACK, _PK * _HID), jnp.bfloat16),
)


def kernel(x, table, W, b):
    h = _gather(x.astype(jnp.int32), table)
    hp = h.reshape(_NPACK, _PK * _HID)
    wt = W.T
    z = jnp.zeros_like(wt)
    w4 = jnp.block(
        [[wt if i == j else z for j in range(_PK)] for i in range(_PK)]
    )
    b4 = jnp.tile(b, _PK).reshape(1, _PK * _HID)
    out = _proj(hp, w4, b4)
    return out.reshape(_BATCH, _HIST, _HID)


# speedup vs baseline: 1.1662x; 1.0019x over previous
import jax
import jax.numpy as jnp
from jax import lax
from jax.experimental import pallas as pl
from jax.experimental.pallas import tpu as pltpu
from jax.experimental.pallas import tpu_sc as plsc

_VOCAB = 1000000
_HID = 64
_BATCH = 4096
_HIST = 200
_NUM_IDX = _BATCH * _HIST
_SPLITS = ((0, 104), (104, 96))
_NC, _NS = 2, 16
_NW = _NC * _NS
_ROWS_W = _BATCH // _NW
_BB = 8
_BROWS = _BB * _HIST
_NBURST = _ROWS_W // _BB
_NBUF = 2


def _gather_body(idx_hbm, tab_hbm, out_hbm, idx_v, rows_v, isem, gsem, wsem):
    wid = lax.axis_index("s") * _NC + lax.axis_index("c")
    base = wid * _ROWS_W

    def idx_copy(b, bb0):
        off = pl.multiple_of(base + bb0, _BB)
        return pltpu.make_async_copy(
            idx_hbm.at[pl.ds(off, _BB)], idx_v.at[b], isem.at[b]
        )

    def wb_copy(b, bb0):
        off = pl.multiple_of((base + bb0) * _HIST, _BROWS)
        return pltpu.make_async_copy(
            rows_v.at[b], out_hbm.at[pl.ds(off, _BROWS)], wsem.at[b]
        )

    def gather_copy(b, r, half):
        lo, n = _SPLITS[half]
        return pltpu.make_async_copy(
            tab_hbm.at[idx_v.at[b, r, pl.ds(lo, n)]],
            rows_v.at[b, pl.ds(r * _HIST + lo, n)],
            gsem,
        )

    for b in range(_NBUF):
        idx_copy(b, b * _BB).start()

    def burst_pair(i, carry):
        for b in range(_NBUF):
            bb0 = (i * _NBUF + b) * _BB
            @pl.when(i > 0)
            def _():
                wb_copy(b, bb0).wait()

            idx_copy(b, bb0).wait()
            for r in range(_BB):
                gather_copy(b, r, 0).start()
                gather_copy(b, r, 1).start()
            for r in range(_BB):
                gather_copy(b, r, 0).wait()
                gather_copy(b, r, 1).wait()
            @pl.when(i + 1 < _NBURST // _NBUF)
            def _():
                idx_copy(b, bb0 + _NBUF * _BB).start()

            wb_copy(b, bb0).start()
        return carry

    lax.fori_loop(0, _NBURST // _NBUF, burst_pair, 0)
    last = (_NBURST - _NBUF) * _BB
    for b in range(_NBUF):
        wb_copy(b, last + b * _BB).wait()


_gather = pl.kernel(
    _gather_body,
    out_type=jax.ShapeDtypeStruct((_NUM_IDX, _HID), jnp.bfloat16),
    mesh=plsc.VectorSubcoreMesh(core_axis_name="c", subcore_axis_name="s"),
    scratch_types=[
        pltpu.VMEM((_NBUF, _BB, _HIST), jnp.int32),
        pltpu.VMEM((_NBUF, _BROWS, _HID), jnp.bfloat16),
        pltpu.SemaphoreType.DMA((_NBUF,)),
        pltpu.SemaphoreType.DMA,
        pltpu.SemaphoreType.DMA((_NBUF,)),
    ],
    compiler_params=pltpu.CompilerParams(use_tc_tiling_on_sc=False),
)


_PK = 4
_NPACK = _NUM_IDX // _PK
_BLKP = 12800


def _proj_body(h_ref, w_ref, b_ref, out_ref):
    acc = lax.dot_general(
        h_ref[...], w_ref[...], (((1,), (0,)), ((), ())),
        preferred_element_type=jnp.float32,
    )
    out_ref[...] = (acc + b_ref[...].astype(jnp.float32)).astype(jnp.bfloat16)


_proj = pl.pallas_call(
    _proj_body,
    grid=(_NPACK // _BLKP,),
    in_specs=[
        pl.BlockSpec((_BLKP, _PK * _HID), lambda i: (i, 0)),
        pl.BlockSpec((_PK * _HID, _PK * _HID), lambda i: (0, 0)),
        pl.BlockSpec((1, _PK * _HID), lambda i: (0, 0)),
    ],
    out_specs=pl.BlockSpec((_BLKP, _PK * _HID), lambda i: (i, 0)),
    out_shape=jax.ShapeDtypeStruct((_NPACK, _PK * _HID), jnp.bfloat16),
)


def kernel(x, table, W, b):
    h = _gather(x.astype(jnp.int32), table)
    hp = h.reshape(_NPACK, _PK * _HID)
    wt = W.T
    z = jnp.zeros_like(wt)
    w4 = jnp.block(
        [[wt if i == j else z for j in range(_PK)] for i in range(_PK)]
    )
    b4 = jnp.tile(b, _PK).reshape(1, _PK * _HID)
    out = _proj(hp, w4, b4)
    return out.reshape(_BATCH, _HIST, _HID)
